# baseline (speedup 1.0000x reference)
import functools

import jax
import jax.numpy as jnp
from jax.experimental import pallas as pl
from jax.experimental.pallas import tpu as pltpu

EMB = 300
HEADS = 8
HEAD_DIM = 32
HD = HEADS * HEAD_DIM
BATCH = 4
SEQ = 512
MEM = 32768
KNN_K = 32
SCALE = HEAD_DIM ** -0.5
NEG_INF = float("-inf")


def _proj_attn_body(x_ref, wq_ref, bq_ref, wk_ref, bk_ref, wv_ref, bv_ref,
                    q_ref, k_ref, v_ref, o_ref):
    x = x_ref[0]
    q = jnp.dot(x, wq_ref[...], preferred_element_type=jnp.float32) + bq_ref[...]
    k = jnp.dot(x, wk_ref[...], preferred_element_type=jnp.float32) + bk_ref[...]
    v = jnp.dot(x, wv_ref[...], preferred_element_type=jnp.float32) + bv_ref[...]
    q_ref[0] = q
    k_ref[0] = k
    v_ref[0] = v

    rows = jax.lax.broadcasted_iota(jnp.int32, (SEQ, SEQ), 0)
    cols = jax.lax.broadcasted_iota(jnp.int32, (SEQ, SEQ), 1)
    causal = cols > rows

    outs = []
    for h in range(HEADS):
        sl = slice(h * HEAD_DIM, (h + 1) * HEAD_DIM)
        qh = q[:, sl]
        kh = k[:, sl]
        vh = v[:, sl]
        qk = jax.lax.dot_general(qh, kh, (((1,), (1,)), ((), ())),
                                 preferred_element_type=jnp.float32) * SCALE
        qk = jnp.where(causal, NEG_INF, qk)
        m = jnp.max(qk, axis=1, keepdims=True)
        p = jnp.exp(qk - m)
        attn = p / jnp.sum(p, axis=1, keepdims=True)
        outs.append(jnp.dot(attn, vh, preferred_element_type=jnp.float32))
    o_ref[0] = jnp.concatenate(outs, axis=1)


def _scores_body(q_ref, mk_ref, s_ref):
    s_ref[0] = jax.lax.dot_general(
        q_ref[0], mk_ref[...], (((1,), (1,)), ((), ())),
        preferred_element_type=jnp.float32)


def _ext_body(qd_ref, q_ref, ke_ref, ve_ref, g_ref, wo_ref, bo_ref, out_ref):
    qd = qd_ref[0]
    q = q_ref[0]
    ke = ke_ref[0]
    ve = ve_ref[0]
    g = jax.nn.sigmoid(g_ref[...])
    outs = []
    for h in range(HEADS):
        sl = slice(h * HEAD_DIM, (h + 1) * HEAD_DIM)
        qh = q[:, sl]
        keh = ke[:, :, sl]
        veh = ve[:, :, sl]
        logits = jnp.sum(qh[:, None, :] * keh, axis=2) * SCALE
        m = jnp.max(logits, axis=1, keepdims=True)
        p = jnp.exp(logits - m)
        attn = p / jnp.sum(p, axis=1, keepdims=True)
        oh = jnp.sum(attn[:, :, None] * veh, axis=1)
        gh = g[h]
        outs.append(qd[:, sl] * gh + oh * (1.0 - gh))
    qkv = jnp.concatenate(outs, axis=1)
    out_ref[0] = jnp.dot(qkv, wo_ref[...], preferred_element_type=jnp.float32) \
        + bo_ref[...]


M_BLK = 4096
S_BLK = 256


@jax.jit
def kernel(input, Wq, bq, Wk, bk, Wv, bv, Wo, bo, gate, memory_keys,
           memory_values):
    f32 = jnp.float32
    bq2 = bq.reshape(1, HD)
    bk2 = bk.reshape(1, HD)
    bv2 = bv.reshape(1, HD)
    bo2 = bo.reshape(1, EMB)

    q_flat, k_flat, v_flat, qkv_dense = pl.pallas_call(
        _proj_attn_body,
        grid=(BATCH,),
        in_specs=[
            pl.BlockSpec((1, SEQ, EMB), lambda b: (b, 0, 0)),
            pl.BlockSpec((EMB, HD), lambda b: (0, 0)),
            pl.BlockSpec((1, HD), lambda b: (0, 0)),
            pl.BlockSpec((EMB, HD), lambda b: (0, 0)),
            pl.BlockSpec((1, HD), lambda b: (0, 0)),
            pl.BlockSpec((EMB, HD), lambda b: (0, 0)),
            pl.BlockSpec((1, HD), lambda b: (0, 0)),
        ],
        out_specs=[
            pl.BlockSpec((1, SEQ, HD), lambda b: (b, 0, 0)),
            pl.BlockSpec((1, SEQ, HD), lambda b: (b, 0, 0)),
            pl.BlockSpec((1, SEQ, HD), lambda b: (b, 0, 0)),
            pl.BlockSpec((1, SEQ, HD), lambda b: (b, 0, 0)),
        ],
        out_shape=[jax.ShapeDtypeStruct((BATCH, SEQ, HD), f32)] * 4,
    )(input, Wq, bq2, Wk, bk2, Wv, bv2)

    scores = pl.pallas_call(
        _scores_body,
        grid=(BATCH, MEM // M_BLK),
        in_specs=[
            pl.BlockSpec((1, SEQ, HD), lambda b, m: (b, 0, 0)),
            pl.BlockSpec((M_BLK, HD), lambda b, m: (m, 0)),
        ],
        out_specs=pl.BlockSpec((1, SEQ, M_BLK), lambda b, m: (b, 0, m)),
        out_shape=jax.ShapeDtypeStruct((BATCH, SEQ, MEM), f32),
    )(q_flat, memory_keys)

    _, idx = jax.lax.top_k(scores, KNN_K)
    k_ext = jnp.take(memory_keys, idx, axis=0)
    v_ext = jnp.take(memory_values, idx, axis=0)

    output = pl.pallas_call(
        _ext_body,
        grid=(BATCH, SEQ // S_BLK),
        in_specs=[
            pl.BlockSpec((1, S_BLK, HD), lambda b, s: (b, s, 0)),
            pl.BlockSpec((1, S_BLK, HD), lambda b, s: (b, s, 0)),
            pl.BlockSpec((1, S_BLK, KNN_K, HD), lambda b, s: (b, s, 0, 0)),
            pl.BlockSpec((1, S_BLK, KNN_K, HD), lambda b, s: (b, s, 0, 0)),
            pl.BlockSpec((HEADS, 1, 1), lambda b, s: (0, 0, 0)),
            pl.BlockSpec((HD, EMB), lambda b, s: (0, 0)),
            pl.BlockSpec((1, EMB), lambda b, s: (0, 0)),
        ],
        out_specs=pl.BlockSpec((1, S_BLK, EMB), lambda b, s: (b, s, 0)),
        out_shape=jax.ShapeDtypeStruct((BATCH, SEQ, EMB), f32),
    )(qkv_dense, q_flat, k_ext, v_ext, gate, Wo, bo2)

    current_kv_memory = jnp.stack((k_flat, v_flat), axis=-2)
    return output, current_kv_memory
